# Initial kernel scaffold; baseline (speedup 1.0000x reference)
import jax
import jax.numpy as jnp
from jax.experimental import pallas as pl


def kernel(net, logits, clean_logits, noise_std):
    raise NotImplementedError("write your pallas kernel here")


# speedup vs baseline: 2.2643x; 2.2643x over previous
import functools

import jax
import jax.numpy as jnp
from jax import lax
from jax.experimental import pallas as pl
from jax.experimental.pallas import tpu as pltpu
from jax.experimental.pallas import tpu_sc as plsc

TOP_K = 2
NUM_EXPERTS = 16
N_TOK = 8192
D_MODEL = 2048

TBLK = 512
NBLK = N_TOK // TBLK

NC, NS, LANES = 2, 16, 16
NW = NC * NS
TOK_PER_W = N_TOK // NW
CH = 16
NCHUNK = TOK_PER_W // CH


def _tc_route_body(logits_ref, clean_ref, nstd_ref,
                   gates_ref, load_ref, part_ref, hist_ref, start_ref,
                   rlo_ref, rhi_ref, elo_ref, ehi_ref, glo_ref, ghi_ref):
    i = pl.program_id(0)

    @pl.when(i == 0)
    def _init():
        load_ref[...] = jnp.zeros((1, NUM_EXPERTS), jnp.float32)
        part_ref[...] = jnp.zeros((1, NUM_EXPERTS), jnp.int32)
        hist_ref[...] = jnp.zeros((1, NUM_EXPERTS), jnp.int32)

    logits = logits_ref[...]
    iota_e = lax.broadcasted_iota(jnp.int32, (TBLK, NUM_EXPERTS), 1)
    neginf = jnp.float32(-jnp.inf)

    m1 = jnp.max(logits, axis=1, keepdims=True)
    i1 = jnp.min(jnp.where(logits == m1, iota_e, NUM_EXPERTS), axis=1)
    l2 = jnp.where(iota_e == i1[:, None], neginf, logits)
    m2 = jnp.max(l2, axis=1, keepdims=True)
    i2 = jnp.min(jnp.where(l2 == m2, iota_e, NUM_EXPERTS), axis=1)
    l3 = jnp.where(iota_e == i2[:, None], neginf, l2)
    m3 = jnp.max(l3, axis=1, keepdims=True)

    e2v = jnp.exp(m2[:, 0] - m1[:, 0])
    g1 = 1.0 / (1.0 + e2v)
    g2 = e2v / (1.0 + e2v)

    oh1 = (iota_e == i1[:, None]).astype(jnp.float32)
    oh2 = (iota_e == i2[:, None]).astype(jnp.float32)
    gates_ref[...] = g1[:, None] * oh1 + g2[:, None] * oh2
    part_blk = ((g1 > 0).astype(jnp.float32)[:, None] * oh1
                + (g2 > 0).astype(jnp.float32)[:, None] * oh2).sum(axis=0)
    part_ref[...] += part_blk.astype(jnp.int32)[None, :]

    clean = clean_ref[...]
    nstd = nstd_ref[...]
    inv_sqrt2 = jnp.float32(0.7071067811865476)
    prob_in = 0.5 * (1.0 + lax.erf((clean - m3) / nstd * inv_sqrt2))
    prob_out = 0.5 * (1.0 + lax.erf((clean - m2) / nstd * inv_sqrt2))
    prob = jnp.where(logits > m3, prob_in, prob_out)
    load_ref[...] += prob.sum(axis=0)[None, :]

    e_lo = jnp.minimum(i1, i2)
    e_hi = jnp.maximum(i1, i2)
    swap = i1 < i2
    g_lo = jnp.where(swap, g1, g2)
    g_hi = jnp.where(swap, g2, g1)
    ohlo = (iota_e == e_lo[:, None]).astype(jnp.float32)
    ohhi = (iota_e == e_hi[:, None]).astype(jnp.float32)
    pair_oh = ohlo + ohhi

    rowi = lax.broadcasted_iota(jnp.int32, (TBLK, TBLK), 0)
    colj = lax.broadcasted_iota(jnp.int32, (TBLK, TBLK), 1)
    tril = (colj < rowi).astype(jnp.float32)
    excl = jax.lax.dot(tril, pair_oh, preferred_element_type=jnp.float32)

    carry = hist_ref[0, :].astype(jnp.float32)
    rank_lo = (excl * ohlo).sum(axis=1) + (ohlo * carry[None, :]).sum(axis=1)
    rank_hi = (excl * ohhi).sum(axis=1) + (ohhi * carry[None, :]).sum(axis=1)
    hist_ref[...] += pair_oh.sum(axis=0).astype(jnp.int32)[None, :]

    rlo_ref[...] = rank_lo.astype(jnp.int32)
    rhi_ref[...] = rank_hi.astype(jnp.int32)
    elo_ref[...] = e_lo
    ehi_ref[...] = e_hi
    glo_ref[...] = g_lo
    ghi_ref[...] = g_hi

    @pl.when(i == NBLK - 1)
    def _start():
        h = hist_ref[...]
        zc = jnp.zeros((1, NUM_EXPERTS), jnp.int32)
        c = h
        for s in (1, 2, 4, 8):
            shifted = jnp.concatenate([zc[:, :s], c[:, :-s]], axis=1)
            c = c + shifted
        start_ref[...] = c - h


def _tc_route(logits, clean_logits, noise_std):
    out_shape = (
        jax.ShapeDtypeStruct((N_TOK, NUM_EXPERTS), jnp.float32),
        jax.ShapeDtypeStruct((1, NUM_EXPERTS), jnp.float32),
        jax.ShapeDtypeStruct((1, NUM_EXPERTS), jnp.int32),
        jax.ShapeDtypeStruct((1, NUM_EXPERTS), jnp.int32),
        jax.ShapeDtypeStruct((1, NUM_EXPERTS), jnp.int32),
        jax.ShapeDtypeStruct((N_TOK,), jnp.int32),
        jax.ShapeDtypeStruct((N_TOK,), jnp.int32),
        jax.ShapeDtypeStruct((N_TOK,), jnp.int32),
        jax.ShapeDtypeStruct((N_TOK,), jnp.int32),
        jax.ShapeDtypeStruct((N_TOK,), jnp.float32),
        jax.ShapeDtypeStruct((N_TOK,), jnp.float32),
    )
    blk_tok = pl.BlockSpec((TBLK, NUM_EXPERTS), lambda i: (i, 0))
    blk_one = pl.BlockSpec((1, NUM_EXPERTS), lambda i: (0, 0))
    blk_vec = pl.BlockSpec((TBLK,), lambda i: (i,))
    return pl.pallas_call(
        _tc_route_body,
        grid=(NBLK,),
        in_specs=[blk_tok, blk_tok, blk_tok],
        out_specs=(blk_tok, blk_one, blk_one, blk_one, blk_one,
                   blk_vec, blk_vec, blk_vec, blk_vec, blk_vec, blk_vec),
        out_shape=out_shape,
    )(logits, clean_logits, noise_std)


def _sc_dispatch_body(net_hbm, start_hbm, rlo_hbm, rhi_hbm, elo_hbm, ehi_hbm,
                      glo_hbm, ghi_hbm,
                      ei_hbm, bi_hbm, gg_hbm,
                      start_v, rlo_v, rhi_v, elo_v, ehi_v,
                      posrow_v, rows_v,
                      allr_v, allr2_v, alle_v, alle2_v, allg_v, allg2_v,
                      dest_bi_v, dest_gg_v,
                      sem0, sem1):
    wid = lax.axis_index("s") * NC + lax.axis_index("c")
    base = wid * TOK_PER_W

    pltpu.sync_copy(start_hbm, start_v)

    pltpu.sync_copy(rlo_hbm.at[pl.ds(base, TOK_PER_W)], rlo_v)
    pltpu.sync_copy(rhi_hbm.at[pl.ds(base, TOK_PER_W)], rhi_v)
    pltpu.sync_copy(elo_hbm.at[pl.ds(base, TOK_PER_W)], elo_v)
    pltpu.sync_copy(ehi_hbm.at[pl.ds(base, TOK_PER_W)], ehi_v)

    for j in range(NCHUNK):
        off = j * CH
        rk = rlo_v[pl.ds(off, LANES)]
        ee = elo_v[pl.ds(off, LANES)]
        posrow_v[2 * j, :] = plsc.load_gather(start_v, [ee]) + rk
        rk = rhi_v[pl.ds(off, LANES)]
        ee = ehi_v[pl.ds(off, LANES)]
        posrow_v[2 * j + 1, :] = plsc.load_gather(start_v, [ee]) + rk

    for j in range(NCHUNK):
        pltpu.sync_copy(net_hbm.at[pl.ds(base + j * CH, CH)], rows_v)
        c0 = pltpu.async_copy(rows_v, ei_hbm.at[posrow_v.at[2 * j]], sem0)
        c1 = pltpu.async_copy(rows_v, ei_hbm.at[posrow_v.at[2 * j + 1]], sem1)
        c0.wait()
        c1.wait()

    @pl.when(wid == 0)
    def _scatter_bi():
        pltpu.sync_copy(rlo_hbm, allr_v)
        pltpu.sync_copy(rhi_hbm, allr2_v)
        pltpu.sync_copy(elo_hbm, alle_v)
        pltpu.sync_copy(ehi_hbm, alle2_v)

        def body(i, carry):
            off = pl.multiple_of(i * LANES, 8)
            tok = jax.lax.iota(jnp.int32, LANES) + i * LANES
            pos = plsc.load_gather(start_v, [alle_v[pl.ds(off, LANES)]]) \
                + allr_v[pl.ds(off, LANES)]
            plsc.store_scatter(dest_bi_v, [pos], tok)
            pos = plsc.load_gather(start_v, [alle2_v[pl.ds(off, LANES)]]) \
                + allr2_v[pl.ds(off, LANES)]
            plsc.store_scatter(dest_bi_v, [pos], tok)
            return carry

        lax.fori_loop(0, N_TOK // LANES, body, 0)
        pltpu.sync_copy(dest_bi_v, bi_hbm)

    @pl.when(wid == 1)
    def _scatter_gg():
        pltpu.sync_copy(rlo_hbm, allr_v)
        pltpu.sync_copy(rhi_hbm, allr2_v)
        pltpu.sync_copy(elo_hbm, alle_v)
        pltpu.sync_copy(ehi_hbm, alle2_v)
        pltpu.sync_copy(glo_hbm, allg_v)
        pltpu.sync_copy(ghi_hbm, allg2_v)

        def body(i, carry):
            off = pl.multiple_of(i * LANES, 8)
            pos = plsc.load_gather(start_v, [alle_v[pl.ds(off, LANES)]]) \
                + allr_v[pl.ds(off, LANES)]
            plsc.store_scatter(dest_gg_v, [pos], allg_v[pl.ds(off, LANES)])
            pos = plsc.load_gather(start_v, [alle2_v[pl.ds(off, LANES)]]) \
                + allr2_v[pl.ds(off, LANES)]
            plsc.store_scatter(dest_gg_v, [pos], allg2_v[pl.ds(off, LANES)])
            return carry

        lax.fori_loop(0, N_TOK // LANES, body, 0)
        pltpu.sync_copy(dest_gg_v, gg_hbm)


def _sc_dispatch(net, start, rlo, rhi, elo, ehi, glo, ghi):
    mesh = plsc.VectorSubcoreMesh(core_axis_name="c", subcore_axis_name="s")
    fn = functools.partial(
        pl.kernel,
        out_type=(
            jax.ShapeDtypeStruct((N_TOK * TOP_K, D_MODEL), jnp.float32),
            jax.ShapeDtypeStruct((N_TOK * TOP_K,), jnp.int32),
            jax.ShapeDtypeStruct((N_TOK * TOP_K,), jnp.float32),
        ),
        mesh=mesh,
        compiler_params=pltpu.CompilerParams(needs_layout_passes=False),
        scratch_types=[
            pltpu.VMEM((NUM_EXPERTS,), jnp.int32),
            pltpu.VMEM((TOK_PER_W,), jnp.int32),
            pltpu.VMEM((TOK_PER_W,), jnp.int32),
            pltpu.VMEM((TOK_PER_W,), jnp.int32),
            pltpu.VMEM((TOK_PER_W,), jnp.int32),
            pltpu.VMEM((2 * NCHUNK, CH), jnp.int32),
            pltpu.VMEM((CH, D_MODEL), jnp.float32),
            pltpu.VMEM((N_TOK,), jnp.int32),
            pltpu.VMEM((N_TOK,), jnp.int32),
            pltpu.VMEM((N_TOK,), jnp.int32),
            pltpu.VMEM((N_TOK,), jnp.int32),
            pltpu.VMEM((N_TOK,), jnp.float32),
            pltpu.VMEM((N_TOK,), jnp.float32),
            pltpu.VMEM((N_TOK * TOP_K,), jnp.int32),
            pltpu.VMEM((N_TOK * TOP_K,), jnp.float32),
            pltpu.SemaphoreType.DMA,
            pltpu.SemaphoreType.DMA,
        ],
    )(_sc_dispatch_body)
    return fn(net, start, rlo, rhi, elo, ehi, glo, ghi)


def kernel(net, logits, clean_logits, noise_std):
    (gates, load11, part11, hist11, start11,
     rlo, rhi, elo, ehi, glo, ghi) = _tc_route(logits, clean_logits, noise_std)
    del hist11
    ei, bi, gg = _sc_dispatch(net, start11.reshape(NUM_EXPERTS),
                              rlo, rhi, elo, ehi, glo, ghi)
    return (gates, ei, bi, gg[:, None],
            load11.reshape(NUM_EXPERTS), part11.reshape(NUM_EXPERTS))
